# baseline (speedup 1.0000x reference)
import functools

import jax
import jax.numpy as jnp
from jax import lax
from jax.experimental import pallas as pl
from jax.experimental.pallas import tpu as pltpu
from jax.experimental.pallas import tpu_sc as plsc

B = 1024
N = 4096
C_OLD = 23
C_NEW = 11

NUM_WORKERS = 32
BATCH_PER_W = B // NUM_WORKERS
CHUNK_ROWS = 2048
CHUNKS = N // CHUNK_ROWS
CHUNK_ELEMS = CHUNK_ROWS * C_OLD
GROUPS_PER_CHUNK = CHUNK_ROWS // 16
OUT_PER_W = BATCH_PER_W * C_NEW * 16


def _sc_grouped_sums(x):
    mesh = plsc.VectorSubcoreMesh(core_axis_name="c", subcore_axis_name="s")

    @functools.partial(
        pl.kernel,
        mesh=mesh,
        out_type=jax.ShapeDtypeStruct((NUM_WORKERS, OUT_PER_W), jnp.float32),
        scratch_types=[
            pltpu.VMEM((CHUNK_ELEMS,), jnp.float32),
            pltpu.VMEM((OUT_PER_W,), jnp.float32),
        ],
        compiler_params=pltpu.CompilerParams(needs_layout_passes=False),
    )
    def k(x_hbm, out_hbm, buf, outv):
        wid = lax.axis_index("s") * 2 + lax.axis_index("c")
        iota = lax.iota(jnp.int32, 16)
        lane_base = iota * C_OLD

        def group_body(t, accs):
            idx0 = lane_base + t * (16 * C_OLD)
            cols = [plsc.load_gather(buf, [idx0 + k_]) for k_ in range(C_OLD)]
            gsums = [cols[2 * g] + cols[2 * g + 1] for g in range(C_NEW - 1)]
            gsums.append(cols[20] + cols[21] + cols[22])
            s = gsums[0]
            for g in range(1, C_NEW):
                s = s + gsums[g]
            w = 1.0 / s
            return tuple(accs[g] + gsums[g] * w for g in range(C_NEW))

        def batch_body(j, _):
            accs = tuple(jnp.zeros((16,), jnp.float32) for _ in range(C_NEW))
            for c in range(CHUNKS):
                pltpu.sync_copy(x_hbm.at[wid, j, c], buf)
                accs = lax.fori_loop(0, GROUPS_PER_CHUNK, group_body, accs)
            for g in range(C_NEW):
                start = pl.multiple_of((j * C_NEW + g) * 16, 16)
                outv[pl.ds(start, 16)] = accs[g]
            return 0

        lax.fori_loop(0, BATCH_PER_W, batch_body, 0)
        pltpu.sync_copy(outv, out_hbm.at[wid])

    return k(x)


def _tc_kl_loss(v, targets):
    def body(v_ref, t_ref, o_ref):
        t = t_ref[...]
        ap = jnp.sum(v_ref[...], axis=-1) * (1.0 / N)
        pw = t * (jnp.log(t) - jnp.log(ap))
        o_ref[0, 0] = jnp.sum(pw) * (1.0 / B)

    out = pl.pallas_call(
        body,
        out_shape=jax.ShapeDtypeStruct((1, 1), jnp.float32),
        out_specs=pl.BlockSpec(memory_space=pltpu.SMEM),
    )(v, targets)
    return out[0, 0]


@jax.jit
def kernel(inputs, targets):
    x = inputs.reshape(NUM_WORKERS, BATCH_PER_W, CHUNKS, CHUNK_ELEMS)
    v = _sc_grouped_sums(x).reshape(B, C_NEW, 16)
    return _tc_kl_loss(v, targets)
